# baseline (speedup 1.0000x reference)
import functools

import jax
import jax.numpy as jnp
from jax.experimental import pallas as pl
from jax.experimental.pallas import tpu as pltpu


_TILE_B = 4096


def _mlp_fused_kernel(xa_ref, xb_ref, w1a_ref, w1b_ref, b1_ref,
                      w2_ref, b2_ref, w3_ref, b3_ref,
                      w4_ref, b4_ref, o_ref):
    h = jnp.dot(xa_ref[...], w1a_ref[...], preferred_element_type=jnp.float32)
    h = h + jnp.dot(xb_ref[...], w1b_ref[...], preferred_element_type=jnp.float32)
    h = jnp.maximum(h + b1_ref[...], 0.0)
    h = jnp.dot(h, w2_ref[...], preferred_element_type=jnp.float32)
    h = jnp.maximum(h + b2_ref[...], 0.0)
    h = jnp.dot(h, w3_ref[...], preferred_element_type=jnp.float32)
    h = jnp.maximum(h + b3_ref[...], 0.0)
    logit = jnp.sum(h * w4_ref[...], axis=1, keepdims=True) + b4_ref[...]
    o_ref[...] = jax.nn.sigmoid(logit)


@functools.partial(jax.jit, static_argnames=("tile_b",))
def _mlp_forward(x, w1, b1, w2, b2, w3, b3, w4, b4, tile_b=_TILE_B):
    batch, n_in = x.shape
    num_tiles = pl.cdiv(batch, tile_b)
    padded = num_tiles * tile_b
    if padded != batch:
        x = jnp.pad(x, ((0, padded - batch), (0, 0)))

    w4r = w4.T
    half = n_in // 2
    w1a, w1b = w1[:half], w1[half:]

    const = lambda i: (0, 0)
    resident = lambda a: pl.BlockSpec(a.shape, const)

    out = pl.pallas_call(
        _mlp_fused_kernel,
        out_shape=jax.ShapeDtypeStruct((padded, 1), jnp.float32),
        grid=(num_tiles,),
        in_specs=[
            pl.BlockSpec((tile_b, half), lambda i: (i, 0)),
            pl.BlockSpec((tile_b, half), lambda i: (i, 1)),
            resident(w1a), resident(w1b), resident(b1),
            resident(w2), resident(b2),
            resident(w3), resident(b3),
            resident(w4r), resident(b4),
        ],
        out_specs=pl.BlockSpec((tile_b, 1), lambda i: (i, 0)),
        compiler_params=pltpu.CompilerParams(
            dimension_semantics=("parallel",),
        ),
    )(x, x, w1a, w1b, b1, w2, b2, w3, b3, w4r, b4)

    return out[:batch]


def kernel(x, w1, b1, w2, b2, w3, b3, w4, b4):
    return _mlp_forward(x, w1, b1, w2, b2, w3, b3, w4, b4)


# speedup vs baseline: 1.0433x; 1.0433x over previous
import functools

import jax
import jax.numpy as jnp
from jax.experimental import pallas as pl
from jax.experimental.pallas import tpu as pltpu


_TILE_B = 8192


def _mlp_fused_kernel(xa_ref, xb_ref, w1a_ref, w1b_ref, b1_ref,
                      w2_ref, b2_ref, w3_ref, b3_ref,
                      w4_ref, b4_ref, o_ref):
    xa = xa_ref[...].astype(jnp.bfloat16)
    xb = xb_ref[...].astype(jnp.bfloat16)
    h = jnp.dot(xa, w1a_ref[...], preferred_element_type=jnp.float32)
    h = h + jnp.dot(xb, w1b_ref[...], preferred_element_type=jnp.float32)
    h = jnp.maximum(h + b1_ref[...], 0.0)
    h = jnp.dot(h, w2_ref[...], preferred_element_type=jnp.float32)
    h = jnp.maximum(h + b2_ref[...], 0.0)
    h = jnp.dot(h, w3_ref[...], preferred_element_type=jnp.float32)
    h = jnp.maximum(h + b3_ref[...], 0.0)
    logit = jnp.sum(h * w4_ref[...], axis=1, keepdims=True) + b4_ref[...]
    o_ref[...] = jax.nn.sigmoid(logit)


@functools.partial(jax.jit, static_argnames=("tile_b",))
def _mlp_forward(x, w1, b1, w2, b2, w3, b3, w4, b4, tile_b=_TILE_B):
    batch, n_in = x.shape
    num_tiles = pl.cdiv(batch, tile_b)
    padded = num_tiles * tile_b
    if padded != batch:
        x = jnp.pad(x, ((0, padded - batch), (0, 0)))

    w4r = w4.T
    half = n_in // 2
    w1a = w1[:half].astype(jnp.bfloat16)
    w1b = w1[half:].astype(jnp.bfloat16)

    const = lambda i: (0, 0)
    resident = lambda a: pl.BlockSpec(a.shape, const)

    out = pl.pallas_call(
        _mlp_fused_kernel,
        out_shape=jax.ShapeDtypeStruct((padded, 1), jnp.float32),
        grid=(num_tiles,),
        in_specs=[
            pl.BlockSpec((tile_b, half), lambda i: (i, 0)),
            pl.BlockSpec((tile_b, half), lambda i: (i, 1)),
            resident(w1a), resident(w1b), resident(b1),
            resident(w2), resident(b2),
            resident(w3), resident(b3),
            resident(w4r), resident(b4),
        ],
        out_specs=pl.BlockSpec((tile_b, 1), lambda i: (i, 0)),
        compiler_params=pltpu.CompilerParams(
            dimension_semantics=("parallel",),
        ),
    )(x, x, w1a, w1b, b1, w2, b2, w3, b3, w4r, b4)

    return out[:batch]


def kernel(x, w1, b1, w2, b2, w3, b3, w4, b4):
    return _mlp_forward(x, w1, b1, w2, b2, w3, b3, w4, b4)


# speedup vs baseline: 1.4331x; 1.3736x over previous
import functools

import jax
import jax.numpy as jnp
from jax.experimental import pallas as pl
from jax.experimental.pallas import tpu as pltpu


_TILE_B = 8192


def _mlp_fused_kernel(xa_ref, xb_ref, w1a_ref, w1b_ref, b1_ref,
                      w2_ref, b2_ref, w3_ref, b3_ref,
                      w4_ref, b4_ref, o_ref):
    xa = xa_ref[...].astype(jnp.bfloat16)
    xb = xb_ref[...].astype(jnp.bfloat16)
    h = jnp.dot(xa, w1a_ref[...], preferred_element_type=jnp.float32)
    h = h + jnp.dot(xb, w1b_ref[...], preferred_element_type=jnp.float32)
    h = jnp.maximum(h + b1_ref[...], 0.0)
    h = jnp.dot(h, w2_ref[...], preferred_element_type=jnp.float32)
    h = jnp.maximum(h + b2_ref[...], 0.0)
    h = jnp.dot(h, w3_ref[...], preferred_element_type=jnp.float32)
    h = jnp.maximum(h + b3_ref[...], 0.0)
    logit = jax.lax.dot_general(
        w4_ref[...], h, (((0,), (1,)), ((), ())),
        preferred_element_type=jnp.float32)
    o_ref[...] = jax.nn.sigmoid(logit + b4_ref[...])


@functools.partial(jax.jit, static_argnames=("tile_b",))
def _mlp_forward(x, w1, b1, w2, b2, w3, b3, w4, b4, tile_b=_TILE_B):
    batch, n_in = x.shape
    num_tiles = pl.cdiv(batch, tile_b)
    padded = num_tiles * tile_b
    if padded != batch:
        x = jnp.pad(x, ((0, padded - batch), (0, 0)))

    half = n_in // 2
    w1a = w1[:half].astype(jnp.bfloat16)
    w1b = w1[half:].astype(jnp.bfloat16)

    const = lambda i: (0, 0)
    resident = lambda a: pl.BlockSpec(a.shape, const)

    out = pl.pallas_call(
        _mlp_fused_kernel,
        out_shape=jax.ShapeDtypeStruct((1, padded), jnp.float32),
        grid=(num_tiles,),
        in_specs=[
            pl.BlockSpec((tile_b, half), lambda i: (i, 0)),
            pl.BlockSpec((tile_b, half), lambda i: (i, 1)),
            resident(w1a), resident(w1b), resident(b1),
            resident(w2), resident(b2),
            resident(w3), resident(b3),
            resident(w4), resident(b4),
        ],
        out_specs=pl.BlockSpec((1, tile_b), lambda i: (0, i)),
        compiler_params=pltpu.CompilerParams(
            dimension_semantics=("parallel",),
        ),
    )(x, x, w1a, w1b, b1, w2, b2, w3, b3, w4, b4)

    return out[0, :batch].reshape(batch, 1)


def kernel(x, w1, b1, w2, b2, w3, b3, w4, b4):
    return _mlp_forward(x, w1, b1, w2, b2, w3, b3, w4, b4)
